# baseline (speedup 1.0000x reference)
import functools

import jax
import jax.numpy as jnp
from jax import lax
from jax.experimental import pallas as pl
from jax.experimental.pallas import tpu as pltpu
from jax.experimental.pallas import tpu_sc as plsc

B = 1024
F = 64
WIN = 128
NBUF = 4
NC = 2
NS = 16
NW = NC * NS
BPW = 32
S_TC = 256
S_SC = B - S_TC
SC_WORKERS = S_SC // BPW
TCBUF = 4

_mesh = plsc.VectorSubcoreMesh(core_axis_name="c", subcore_axis_name="s")

_GATHER_DN = lax.GatherDimensionNumbers(
    offset_dims=(), collapsed_slice_dims=(0,), start_index_map=(0,))


def _permute(x, idx):
    return lax.gather(x, idx[:, None], _GATHER_DN, (1,),
                      mode=lax.GatherScatterMode.PROMISE_IN_BOUNDS)


@functools.partial(
    pl.kernel,
    mesh=_mesh,
    out_type=[
        jax.ShapeDtypeStruct((B,), jnp.float32),
        jax.ShapeDtypeStruct((B,), jnp.float32),
    ],
    scratch_types=[
        pltpu.VMEM((BPW,), jnp.int32),
        pltpu.VMEM((BPW,), jnp.int32),
        pltpu.VMEM((NBUF, F, WIN), jnp.float32),
        pltpu.VMEM((NBUF, F, WIN), jnp.float32),
        pltpu.VMEM((BPW,), jnp.float32),
        pltpu.VMEM((BPW,), jnp.float32),
        pltpu.VMEM((BPW,), jnp.float32),
        pltpu.VMEM((BPW,), jnp.float32),
        pltpu.SemaphoreType.DMA,
        pltpu.SemaphoreType.DMA,
    ],
    compiler_params=pltpu.CompilerParams(needs_layout_passes=False),
)
def _sc_gather_dot(users_hbm, items_hbm, uembt_hbm, iembt_hbm, ub_hbm, ib_hbm,
                   r_hbm, d_hbm,
                   uidx, iidx, ublk, iblk, ub, ib, rloc, dloc, sem, bsem):
    wid = lax.axis_index("s") * NC + lax.axis_index("c")

    @pl.when(wid < SC_WORKERS)
    def _():
        base = wid * BPW
        pltpu.sync_copy(users_hbm.at[pl.ds(base, BPW)], uidx)
        pltpu.sync_copy(items_hbm.at[pl.ds(base, BPW)], iidx)
        cpu_b = pltpu.async_copy(ub_hbm.at[0].at[uidx], ub, bsem)
        cpi_b = pltpu.async_copy(ib_hbm.at[0].at[iidx], ib, bsem)
        iota = lax.iota(jnp.int32, 16)
        zeros = jnp.zeros((16,), jnp.int32)
        uvecs = [uidx[pl.ds(0, 16)], uidx[pl.ds(16, 16)]]
        ivecs = [iidx[pl.ds(0, 16)], iidx[pl.ds(16, 16)]]

        def issue(j):
            ru = uvecs[j // 16][j % 16]
            ri = ivecs[j // 16][j % 16]
            off_u = pl.multiple_of(ru & -WIN, WIN)
            off_i = pl.multiple_of(ri & -WIN, WIN)
            s = j % NBUF
            return (pltpu.async_copy(uembt_hbm.at[:, pl.ds(off_u, WIN)],
                                     ublk.at[s], sem),
                    pltpu.async_copy(iembt_hbm.at[:, pl.ds(off_i, WIN)],
                                     iblk.at[s], sem))

        pending = [issue(j) for j in range(NBUF - 1)]
        dvecs = [jnp.zeros((16,), jnp.float32), jnp.zeros((16,), jnp.float32)]
        for j in range(BPW):
            if j + NBUF - 1 < BPW:
                pending.append(issue(j + NBUF - 1))
            for cp in pending[j]:
                cp.wait()
            s = j % NBUF
            cu = zeros + (uvecs[j // 16][j % 16] & (WIN - 1))
            ci = zeros + (ivecs[j // 16][j % 16] & (WIN - 1))
            acc = jnp.zeros((16,), jnp.float32)
            for k in range(F // 16):
                rows = iota + k * 16
                acc = acc + (plsc.load_gather(ublk.at[s], [rows, cu])
                             * plsc.load_gather(iblk.at[s], [rows, ci]))
            for sh in (8, 4, 2, 1):
                acc = acc + _permute(acc, iota ^ sh)
            dvecs[j // 16] = jnp.where(iota == (j % 16), acc, dvecs[j // 16])
        dloc[pl.ds(0, 16)] = dvecs[0]
        dloc[pl.ds(16, 16)] = dvecs[1]
        cpu_b.wait()
        cpi_b.wait()
        for g in range(BPW // 16):
            rloc[pl.ds(g * 16, 16)] = (ub[pl.ds(g * 16, 16)]
                                       + ib[pl.ds(g * 16, 16)])
        pltpu.sync_copy(rloc, r_hbm.at[pl.ds(base, BPW)])
        pltpu.sync_copy(dloc, d_hbm.at[pl.ds(base, BPW)])


def _tc_gather_body(users_ref, items_ref, ut_ref, it_ref, ub_ref, ib_ref,
                    dtc_ref, rtc_ref, ublk, iblk, ubb, ibb, sems):
    iota_w = lax.broadcasted_iota(jnp.int32, (F, WIN), 1)
    iota_b = lax.broadcasted_iota(jnp.int32, (1, WIN), 1)
    iota_n = lax.iota(jnp.int32, S_TC)

    def get_idx(j):
        jc = jnp.minimum(j, S_TC - 1)
        return users_ref[S_SC + jc], items_ref[S_SC + jc]

    def issue(j, s):
        ru, ri = get_idx(j)
        off_u = pl.multiple_of(ru & -WIN, WIN)
        off_i = pl.multiple_of(ri & -WIN, WIN)
        pltpu.make_async_copy(ut_ref.at[:, pl.ds(off_u, WIN)],
                              ublk.at[s], sems.at[s]).start()
        pltpu.make_async_copy(it_ref.at[:, pl.ds(off_i, WIN)],
                              iblk.at[s], sems.at[s]).start()
        pltpu.make_async_copy(ub_ref.at[:, pl.ds(off_u, WIN)],
                              ubb.at[s], sems.at[s]).start()
        pltpu.make_async_copy(ib_ref.at[:, pl.ds(off_i, WIN)],
                              ibb.at[s], sems.at[s]).start()

    def drain(s):
        pltpu.make_async_copy(ut_ref.at[:, pl.ds(0, WIN)],
                              ublk.at[s], sems.at[s]).wait()
        pltpu.make_async_copy(it_ref.at[:, pl.ds(0, WIN)],
                              iblk.at[s], sems.at[s]).wait()
        pltpu.make_async_copy(ub_ref.at[:, pl.ds(0, WIN)],
                              ubb.at[s], sems.at[s]).wait()
        pltpu.make_async_copy(ib_ref.at[:, pl.ds(0, WIN)],
                              ibb.at[s], sems.at[s]).wait()

    for s in range(TCBUF - 1):
        issue(s, s)

    def body(o, carry):
        dvec, rvec = carry
        for b_ in range(TCBUF):
            j = o * TCBUF + b_
            issue(j + TCBUF - 1, (b_ + TCBUF - 1) % TCBUF)
            drain(b_)
            ru, ri = get_idx(j)
            rm_u = ru & (WIN - 1)
            rm_i = ri & (WIN - 1)
            w_i = pltpu.roll(iblk[b_], (rm_u - rm_i) & (WIN - 1), 1)
            p = ublk[b_] * w_i
            dot = jnp.sum(jnp.where(iota_w == rm_u, p, 0.0))
            bu = jnp.sum(jnp.where(iota_b == rm_u, ubb[b_], 0.0))
            bi = jnp.sum(jnp.where(iota_b == rm_i, ibb[b_], 0.0))
            sel = iota_n == j
            dvec = jnp.where(sel, dot, dvec)
            rvec = jnp.where(sel, bu + bi, rvec)
        return dvec, rvec

    z = jnp.zeros((S_TC,), jnp.float32)
    dvec, rvec = lax.fori_loop(0, S_TC // TCBUF, body, (z, z))
    for s in range(TCBUF - 1):
        drain(s)
    dtc_ref[...] = dvec
    rtc_ref[...] = rvec


def _tc_body(r_ref, d_ref, o_ref):
    i = pl.program_id(0)
    t = jnp.transpose(r_ref[...], (1, 0))
    mask = lax.broadcasted_iota(jnp.int32, (128, 8), 1) == i
    rcol = jnp.sum(jnp.where(mask, t, 0.0), axis=1, keepdims=True)
    o_ref[...] = rcol + d_ref[...]


def kernel(users, items, user_emb, item_emb, user_bias, item_bias):
    users = users.astype(jnp.int32)
    items = items.astype(jnp.int32)
    ut, it = user_emb.T, item_emb.T
    ubt, ibt = user_bias.T, item_bias.T
    r_sc, d_sc = _sc_gather_dot(users, items, ut, it, ubt, ibt)
    d_tc, r_tc = pl.pallas_call(
        _tc_gather_body,
        in_specs=[
            pl.BlockSpec(memory_space=pltpu.SMEM),
            pl.BlockSpec(memory_space=pltpu.SMEM),
            pl.BlockSpec(memory_space=pl.ANY),
            pl.BlockSpec(memory_space=pl.ANY),
            pl.BlockSpec(memory_space=pl.ANY),
            pl.BlockSpec(memory_space=pl.ANY),
        ],
        out_shape=[jax.ShapeDtypeStruct((S_TC,), jnp.float32),
                   jax.ShapeDtypeStruct((S_TC,), jnp.float32)],
        scratch_shapes=[
            pltpu.VMEM((TCBUF, F, WIN), jnp.float32),
            pltpu.VMEM((TCBUF, F, WIN), jnp.float32),
            pltpu.VMEM((TCBUF, 1, WIN), jnp.float32),
            pltpu.VMEM((TCBUF, 1, WIN), jnp.float32),
            pltpu.SemaphoreType.DMA((TCBUF,)),
        ],
    )(users, items, ut, it, ubt, ibt)
    d = lax.dynamic_update_slice(d_sc, d_tc, (S_SC,))
    r = lax.dynamic_update_slice(r_sc, r_tc, (S_SC,))
    out = pl.pallas_call(
        _tc_body,
        grid=(8,),
        in_specs=[
            pl.BlockSpec((8, 128), lambda i: (0, 0)),
            pl.BlockSpec((1, B), lambda i: (0, 0)),
        ],
        out_specs=pl.BlockSpec((128, B), lambda i: (i, 0)),
        out_shape=jax.ShapeDtypeStruct((B, B), jnp.float32),
    )(r.reshape(8, 128), d.reshape(1, B))
    return out


# speedup vs baseline: 1.9325x; 1.9325x over previous
import functools

import jax
import jax.numpy as jnp
from jax import lax
from jax.experimental import pallas as pl
from jax.experimental.pallas import tpu as pltpu
from jax.experimental.pallas import tpu_sc as plsc

B = 1024
F = 64
WIN = 128
NBUF = 6
NC = 2
NS = 16
NW = NC * NS
BPW = B // NW

_mesh = plsc.VectorSubcoreMesh(core_axis_name="c", subcore_axis_name="s")

_GATHER_DN = lax.GatherDimensionNumbers(
    offset_dims=(), collapsed_slice_dims=(0,), start_index_map=(0,))


def _permute(x, idx):
    return lax.gather(x, idx[:, None], _GATHER_DN, (1,),
                      mode=lax.GatherScatterMode.PROMISE_IN_BOUNDS)


@functools.partial(
    pl.kernel,
    mesh=_mesh,
    out_type=[
        jax.ShapeDtypeStruct((B,), jnp.float32),
        jax.ShapeDtypeStruct((B,), jnp.float32),
    ],
    scratch_types=[
        pltpu.VMEM((BPW,), jnp.int32),
        pltpu.VMEM((BPW,), jnp.int32),
        pltpu.VMEM((NBUF, F, WIN), jnp.float32),
        pltpu.VMEM((NBUF, F, WIN), jnp.float32),
        pltpu.VMEM((BPW,), jnp.float32),
        pltpu.VMEM((BPW,), jnp.float32),
        pltpu.VMEM((BPW,), jnp.float32),
        pltpu.VMEM((BPW,), jnp.float32),
        pltpu.SemaphoreType.DMA,
        pltpu.SemaphoreType.DMA,
    ],
    compiler_params=pltpu.CompilerParams(needs_layout_passes=False),
)
def _sc_gather_dot(users_hbm, items_hbm, uembt_hbm, iembt_hbm, ub_hbm, ib_hbm,
                   r_hbm, d_hbm,
                   uidx, iidx, ublk, iblk, ub, ib, rloc, dloc, sem, bsem):
    wid = lax.axis_index("s") * NC + lax.axis_index("c")
    base = wid * BPW
    pltpu.sync_copy(users_hbm.at[pl.ds(base, BPW)], uidx)
    pltpu.sync_copy(items_hbm.at[pl.ds(base, BPW)], iidx)
    cpu_b = pltpu.async_copy(ub_hbm.at[0].at[uidx], ub, bsem)
    cpi_b = pltpu.async_copy(ib_hbm.at[0].at[iidx], ib, bsem)
    iota = lax.iota(jnp.int32, 16)
    zeros = jnp.zeros((16,), jnp.int32)
    uvecs = [uidx[pl.ds(0, 16)], uidx[pl.ds(16, 16)]]
    ivecs = [iidx[pl.ds(0, 16)], iidx[pl.ds(16, 16)]]

    def issue(j):
        ru = uvecs[j // 16][j % 16]
        ri = ivecs[j // 16][j % 16]
        off_u = pl.multiple_of(ru & -WIN, WIN)
        off_i = pl.multiple_of(ri & -WIN, WIN)
        s = j % NBUF
        return (pltpu.async_copy(uembt_hbm.at[:, pl.ds(off_u, WIN)],
                                 ublk.at[s], sem),
                pltpu.async_copy(iembt_hbm.at[:, pl.ds(off_i, WIN)],
                                 iblk.at[s], sem))

    pending = [issue(j) for j in range(NBUF - 1)]
    dvecs = [jnp.zeros((16,), jnp.float32), jnp.zeros((16,), jnp.float32)]
    for j in range(BPW):
        if j + NBUF - 1 < BPW:
            pending.append(issue(j + NBUF - 1))
        for cp in pending[j]:
            cp.wait()
        s = j % NBUF
        cu = zeros + (uvecs[j // 16][j % 16] & (WIN - 1))
        ci = zeros + (ivecs[j // 16][j % 16] & (WIN - 1))
        acc = jnp.zeros((16,), jnp.float32)
        for k in range(F // 16):
            rows = iota + k * 16
            acc = acc + (plsc.load_gather(ublk.at[s], [rows, cu])
                         * plsc.load_gather(iblk.at[s], [rows, ci]))
        for sh in (8, 4, 2, 1):
            acc = acc + _permute(acc, iota ^ sh)
        dvecs[j // 16] = jnp.where(iota == (j % 16), acc, dvecs[j // 16])
    dloc[pl.ds(0, 16)] = dvecs[0]
    dloc[pl.ds(16, 16)] = dvecs[1]
    cpu_b.wait()
    cpi_b.wait()
    for g in range(BPW // 16):
        rloc[pl.ds(g * 16, 16)] = (ub[pl.ds(g * 16, 16)]
                                   + ib[pl.ds(g * 16, 16)])
    pltpu.sync_copy(rloc, r_hbm.at[pl.ds(base, BPW)])
    pltpu.sync_copy(dloc, d_hbm.at[pl.ds(base, BPW)])


def _tc_body(r_ref, d_ref, o_ref):
    i = pl.program_id(0)
    t = jnp.transpose(r_ref[...], (1, 0))
    mask = lax.broadcasted_iota(jnp.int32, (128, 8), 1) == i
    rcol = jnp.sum(jnp.where(mask, t, 0.0), axis=1, keepdims=True)
    o_ref[...] = rcol + d_ref[...]


def kernel(users, items, user_emb, item_emb, user_bias, item_bias):
    users = users.astype(jnp.int32)
    items = items.astype(jnp.int32)
    r, d = _sc_gather_dot(users, items, user_emb.T, item_emb.T,
                          user_bias.T, item_bias.T)
    out = pl.pallas_call(
        _tc_body,
        grid=(8,),
        in_specs=[
            pl.BlockSpec((8, 128), lambda i: (0, 0)),
            pl.BlockSpec((1, B), lambda i: (0, 0)),
        ],
        out_specs=pl.BlockSpec((128, B), lambda i: (i, 0)),
        out_shape=jax.ShapeDtypeStruct((B, B), jnp.float32),
    )(r.reshape(8, 128), d.reshape(1, B))
    return out


# speedup vs baseline: 1.9674x; 1.0180x over previous
import functools

import jax
import jax.numpy as jnp
from jax import lax
from jax.experimental import pallas as pl
from jax.experimental.pallas import tpu as pltpu
from jax.experimental.pallas import tpu_sc as plsc

B = 1024
F = 64
WIN = 128
NBUF = 4
NC = 2
NS = 16
NW = NC * NS
BPW = B // NW

_mesh = plsc.VectorSubcoreMesh(core_axis_name="c", subcore_axis_name="s")

_GATHER_DN = lax.GatherDimensionNumbers(
    offset_dims=(), collapsed_slice_dims=(0,), start_index_map=(0,))


def _permute(x, idx):
    return lax.gather(x, idx[:, None], _GATHER_DN, (1,),
                      mode=lax.GatherScatterMode.PROMISE_IN_BOUNDS)


@functools.partial(
    pl.kernel,
    mesh=_mesh,
    out_type=[
        jax.ShapeDtypeStruct((B,), jnp.float32),
        jax.ShapeDtypeStruct((B,), jnp.float32),
    ],
    scratch_types=[
        pltpu.VMEM((BPW,), jnp.int32),
        pltpu.VMEM((BPW,), jnp.int32),
        pltpu.VMEM((NBUF, F, WIN), jnp.float32),
        pltpu.VMEM((NBUF, F, WIN), jnp.float32),
        pltpu.VMEM((BPW,), jnp.float32),
        pltpu.VMEM((BPW,), jnp.float32),
        pltpu.VMEM((BPW,), jnp.float32),
        pltpu.VMEM((BPW,), jnp.float32),
        pltpu.SemaphoreType.DMA,
        pltpu.SemaphoreType.DMA,
    ],
    compiler_params=pltpu.CompilerParams(needs_layout_passes=False),
)
def _sc_gather_dot(users_hbm, items_hbm, uembt_hbm, iembt_hbm, ub_hbm, ib_hbm,
                   r_hbm, d_hbm,
                   uidx, iidx, ublk, iblk, ub, ib, rloc, dloc, sem, bsem):
    wid = lax.axis_index("s") * NC + lax.axis_index("c")
    base = wid * BPW
    pltpu.sync_copy(users_hbm.at[pl.ds(base, BPW)], uidx)
    pltpu.sync_copy(items_hbm.at[pl.ds(base, BPW)], iidx)
    cpu_b = pltpu.async_copy(ub_hbm.at[0].at[uidx], ub, bsem)
    cpi_b = pltpu.async_copy(ib_hbm.at[0].at[iidx], ib, bsem)
    iota = lax.iota(jnp.int32, 16)
    zeros = jnp.zeros((16,), jnp.int32)
    uvecs = [uidx[pl.ds(0, 16)], uidx[pl.ds(16, 16)]]
    ivecs = [iidx[pl.ds(0, 16)], iidx[pl.ds(16, 16)]]

    def issue(j):
        ru = uvecs[j // 16][j % 16]
        ri = ivecs[j // 16][j % 16]
        off_u = pl.multiple_of(ru & -WIN, WIN)
        off_i = pl.multiple_of(ri & -WIN, WIN)
        s = j % NBUF
        return (pltpu.async_copy(uembt_hbm.at[:, pl.ds(off_u, WIN)],
                                 ublk.at[s], sem),
                pltpu.async_copy(iembt_hbm.at[:, pl.ds(off_i, WIN)],
                                 iblk.at[s], sem))

    pending = [issue(j) for j in range(NBUF - 1)]
    dvecs = [jnp.zeros((16,), jnp.float32), jnp.zeros((16,), jnp.float32)]
    for j in range(BPW):
        if j + NBUF - 1 < BPW:
            pending.append(issue(j + NBUF - 1))
        for cp in pending[j]:
            cp.wait()
        s = j % NBUF
        cu = zeros + (uvecs[j // 16][j % 16] & (WIN - 1))
        ci = zeros + (ivecs[j // 16][j % 16] & (WIN - 1))
        acc = jnp.zeros((16,), jnp.float32)
        for k in range(F // 16):
            rows = iota + k * 16
            acc = acc + (plsc.load_gather(ublk.at[s], [rows, cu])
                         * plsc.load_gather(iblk.at[s], [rows, ci]))
        for sh in (8, 4, 2, 1):
            acc = acc + _permute(acc, iota ^ sh)
        dvecs[j // 16] = jnp.where(iota == (j % 16), acc, dvecs[j // 16])
    dloc[pl.ds(0, 16)] = dvecs[0]
    dloc[pl.ds(16, 16)] = dvecs[1]
    cpu_b.wait()
    cpi_b.wait()
    for g in range(BPW // 16):
        rloc[pl.ds(g * 16, 16)] = (ub[pl.ds(g * 16, 16)]
                                   + ib[pl.ds(g * 16, 16)])
    pltpu.sync_copy(rloc, r_hbm.at[pl.ds(base, BPW)])
    pltpu.sync_copy(dloc, d_hbm.at[pl.ds(base, BPW)])


def _tc_body(r_ref, d_ref, o_ref):
    i = pl.program_id(0)
    t = jnp.transpose(r_ref[...], (1, 0))
    mask = lax.broadcasted_iota(jnp.int32, (128, 8), 1) == i
    rcol = jnp.sum(jnp.where(mask, t, 0.0), axis=1, keepdims=True)
    o_ref[...] = rcol + d_ref[...]


def kernel(users, items, user_emb, item_emb, user_bias, item_bias):
    users = users.astype(jnp.int32)
    items = items.astype(jnp.int32)
    r, d = _sc_gather_dot(users, items, user_emb.T, item_emb.T,
                          user_bias.T, item_bias.T)
    out = pl.pallas_call(
        _tc_body,
        grid=(8,),
        in_specs=[
            pl.BlockSpec((8, 128), lambda i: (0, 0)),
            pl.BlockSpec((1, B), lambda i: (0, 0)),
        ],
        out_specs=pl.BlockSpec((128, B), lambda i: (i, 0)),
        out_shape=jax.ShapeDtypeStruct((B, B), jnp.float32),
    )(r.reshape(8, 128), d.reshape(1, B))
    return out
